# baseline (speedup 1.0000x reference)
import jax
import jax.numpy as jnp
from jax.experimental import pallas as pl


def _router_body(x_ref, w_ref, o_ref):
    o_ref[...] = jnp.dot(x_ref[...], w_ref[...],
                         preferred_element_type=jnp.float32)


def kernel(x, W):
    B, S, D = x.shape
    E = W.shape[1]
    M = B * S
    x2 = x.reshape(M, D)
    BLK = 1024
    out = pl.pallas_call(
        _router_body,
        grid=(M // BLK,),
        in_specs=[
            pl.BlockSpec((BLK, D), lambda i: (i, 0)),
            pl.BlockSpec((D, E), lambda i: (0, 0)),
        ],
        out_specs=pl.BlockSpec((BLK, E), lambda i: (i, 0)),
        out_shape=jax.ShapeDtypeStruct((M, E), jnp.float32),
    )(x2, W)
    return out.reshape(B, S, E)


# speedup vs baseline: 1.1938x; 1.1938x over previous
import jax
import jax.numpy as jnp
from jax.experimental import pallas as pl


def _router_body(x_ref, w_ref, o_ref):
    o_ref[...] = jnp.dot(x_ref[...], w_ref[...],
                         preferred_element_type=jnp.float32)


def kernel(x, W):
    B, S, D = x.shape
    E = W.shape[1]
    M = B * S
    x2 = x.reshape(M, D)
    BLK = 4096
    out = pl.pallas_call(
        _router_body,
        grid=(M // BLK,),
        in_specs=[
            pl.BlockSpec((BLK, D), lambda i: (i, 0)),
            pl.BlockSpec((D, E), lambda i: (0, 0)),
        ],
        out_specs=pl.BlockSpec((BLK, E), lambda i: (i, 0)),
        out_shape=jax.ShapeDtypeStruct((M, E), jnp.float32),
    )(x2, W)
    return out.reshape(B, S, E)
